# Initial kernel scaffold; baseline (speedup 1.0000x reference)
import jax
import jax.numpy as jnp
from jax.experimental import pallas as pl


def kernel(private_reserve, move_mask, emb_table):
    raise NotImplementedError("write your pallas kernel here")


# speedup vs baseline: 7.7739x; 7.7739x over previous
import functools

import jax
import jax.numpy as jnp
from jax import lax
from jax.experimental import pallas as pl
from jax.experimental.pallas import tpu as pltpu
from jax.experimental.pallas import tpu_sc as plsc

_B = 16384
_M = 6
_F = 16
_V = 1000
_D = 128
_L = 16
_NMOVE = 4 * _M


@functools.lru_cache(maxsize=None)
def _build_sc_call():
    info = plsc.get_sparse_core_info()
    nc, ns = info.num_cores, info.num_subcores
    nw = nc * ns
    assert _B % (nw * _L) == 0
    bpw = _B // nw
    ngroups = bpw // _L
    prw = bpw * _F * _M
    mkw = bpw * _NMOVE
    rows_per_sub = 64
    vpad = ns * rows_per_sub

    mesh = plsc.VectorSubcoreMesh(core_axis_name="c", subcore_axis_name="s")

    @functools.partial(
        pl.kernel,
        mesh=mesh,
        compiler_params=pltpu.CompilerParams(needs_layout_passes=False),
        out_type=(
            jax.ShapeDtypeStruct((_B * _NMOVE,), jnp.float32),
            jax.ShapeDtypeStruct((_B,), jnp.int32),
        ),
        scratch_types=dict(
            t3_shared=pltpu.VMEM_SHARED((vpad,), jnp.float32),
            row_buf=pltpu.VMEM((rows_per_sub * _D,), jnp.float32),
            t3_part=pltpu.VMEM((rows_per_sub,), jnp.float32),
            t3=pltpu.VMEM((vpad,), jnp.float32),
            pr_v=pltpu.VMEM((prw,), jnp.int32),
            mask_v=pltpu.VMEM((mkw,), jnp.int32),
            out_v=pltpu.VMEM((mkw,), jnp.float32),
            idx_v=pltpu.VMEM((bpw,), jnp.int32),
            sem_pr=pltpu.SemaphoreType.DMA,
            sem_mk=pltpu.SemaphoreType.DMA,
        ),
    )
    def sc_call(pr_hbm, mask_hbm, emb_hbm, bp_hbm, idx_hbm, *, t3_shared,
                row_buf, t3_part, t3, pr_v, mask_v, out_v, idx_v, sem_pr,
                sem_mk):
        c = lax.axis_index("c")
        s = lax.axis_index("s")
        wid = s * nc + c

        cp_pr = pltpu.async_copy(pr_hbm.at[pl.ds(wid * prw, prw)], pr_v,
                                 sem_pr)
        cp_mk = pltpu.async_copy(mask_hbm.at[pl.ds(wid * mkw, mkw)], mask_v,
                                 sem_mk)

        iota = lax.iota(jnp.int32, _L)
        base_row = jnp.minimum(s * rows_per_sub, _V - rows_per_sub)
        pltpu.sync_copy(emb_hbm.at[pl.ds(base_row * _D, rows_per_sub * _D)],
                        row_buf)
        for gg in range(rows_per_sub // _L):
            col3 = plsc.load_gather(row_buf, [(iota + gg * _L) * _D + 3])
            t3_part[pl.ds(gg * _L, _L)] = col3
        pltpu.sync_copy(t3_part, t3_shared.at[pl.ds(base_row, rows_per_sub)])
        plsc.subcore_barrier()
        pltpu.sync_copy(t3_shared, t3)

        cp_pr.wait()
        cp_mk.wait()

        iota_f = iota * (_F * _M)
        iota_n = iota * _NMOVE
        neg1 = jnp.full((_L,), -1.0, jnp.float32)
        neginf = jnp.full((_L,), -jnp.inf, jnp.float32)

        def group(g, carry):
            pr_base = g * (_L * _F * _M)
            mk_base = g * (_L * _NMOVE)
            runmax = neginf
            runidx = jnp.zeros((_L,), jnp.int32)
            for j in range(_NMOVE):
                m, k = divmod(j, 4)
                if k == 0:
                    act = plsc.load_gather(
                        pr_v, [iota_f + (pr_base + m * _F + 1)]) == 1
                off = m * _F + (_F - 8) + 2 * k
                tok = plsc.load_gather(pr_v, [iota_f + (pr_base + off)]) + 1
                tok = jnp.minimum(jnp.maximum(tok, 0), _V - 1)
                bp = plsc.load_gather(t3, [tok])
                mv = plsc.load_gather(mask_v, [iota_n + (mk_base + j)])
                bpm = jnp.where(mv != 0, bp, neg1)
                sc = jnp.where(act, bpm, neginf)
                upd = sc > runmax
                runmax = jnp.where(upd, sc, runmax)
                runidx = jnp.where(upd, jnp.full((_L,), j, jnp.int32), runidx)
                plsc.store_scatter(out_v, [iota_n + (mk_base + j)], bpm)
            idx_v[pl.ds(g * _L, _L)] = runidx
            return carry

        lax.fori_loop(0, ngroups, group, 0)

        pltpu.sync_copy(out_v, bp_hbm.at[pl.ds(wid * mkw, mkw)])
        pltpu.sync_copy(idx_v, idx_hbm.at[pl.ds(wid * bpw, bpw)])

    return sc_call


def kernel(private_reserve, move_mask, emb_table):
    sc_call = _build_sc_call()
    pr_flat = private_reserve.reshape(-1)
    mask_flat = move_mask.astype(jnp.int32).reshape(-1)
    emb_flat = emb_table.reshape(-1)
    bp_flat, index = sc_call(pr_flat, mask_flat, emb_flat)
    return bp_flat.reshape(_B, _M, 4), index


# speedup vs baseline: 10.1568x; 1.3065x over previous
import functools

import jax
import jax.numpy as jnp
from jax import lax
from jax.experimental import pallas as pl
from jax.experimental.pallas import tpu as pltpu
from jax.experimental.pallas import tpu_sc as plsc

_B = 16384
_M = 6
_F = 16
_V = 1000
_D = 128
_L = 16
_NMOVE = 4 * _M


@functools.lru_cache(maxsize=None)
def _build_sc_call():
    info = plsc.get_sparse_core_info()
    nc, ns = info.num_cores, info.num_subcores
    nw = nc * ns
    assert _B % (nw * _L) == 0
    bpw = _B // nw
    ngroups = bpw // _L
    prw = bpw * _F * _M
    mkw = bpw * _NMOVE
    mww = bpw * _M
    rows_per_sub = 64
    vpad = ns * rows_per_sub

    mesh = plsc.VectorSubcoreMesh(core_axis_name="c", subcore_axis_name="s")

    @functools.partial(
        pl.kernel,
        mesh=mesh,
        compiler_params=pltpu.CompilerParams(
            needs_layout_passes=False, disable_bounds_checks=True),
        out_type=(
            jax.ShapeDtypeStruct((_B * _NMOVE,), jnp.float32),
            jax.ShapeDtypeStruct((_B,), jnp.int32),
        ),
        scratch_types=dict(
            t3_shared=pltpu.VMEM_SHARED((vpad,), jnp.float32),
            row_buf=pltpu.VMEM((rows_per_sub * _D,), jnp.float32),
            t3_part=pltpu.VMEM((rows_per_sub,), jnp.float32),
            t3=pltpu.VMEM((vpad,), jnp.float32),
            pr_v=pltpu.VMEM((prw,), jnp.int32),
            mask_v=pltpu.VMEM((mww,), jnp.int32),
            out_v=pltpu.VMEM((mkw,), jnp.float32),
            idx_v=pltpu.VMEM((bpw,), jnp.int32),
            sem_pr=pltpu.SemaphoreType.DMA,
            sem_mk=pltpu.SemaphoreType.DMA,
        ),
    )
    def sc_call(pr_hbm, mask_hbm, emb_hbm, bp_hbm, idx_hbm, *, t3_shared,
                row_buf, t3_part, t3, pr_v, mask_v, out_v, idx_v, sem_pr,
                sem_mk):
        c = lax.axis_index("c")
        s = lax.axis_index("s")
        wid = s * nc + c

        cp_pr = pltpu.async_copy(pr_hbm.at[pl.ds(wid * prw, prw)], pr_v,
                                 sem_pr)
        cp_mk = pltpu.async_copy(mask_hbm.at[pl.ds(wid * mww, mww)], mask_v,
                                 sem_mk)

        iota = lax.iota(jnp.int32, _L)
        base_row = jnp.minimum(s * rows_per_sub, _V - rows_per_sub)
        pltpu.sync_copy(emb_hbm.at[pl.ds(base_row * _D, rows_per_sub * _D)],
                        row_buf)
        for gg in range(rows_per_sub // _L):
            col3 = plsc.load_gather(row_buf, [(iota + gg * _L) * _D + 3])
            t3_part[pl.ds(gg * _L, _L)] = col3
        pltpu.sync_copy(t3_part, t3_shared.at[pl.ds(base_row, rows_per_sub)])
        plsc.subcore_barrier()
        pltpu.sync_copy(t3_shared, t3)

        cp_pr.wait()
        cp_mk.wait()

        iota_f = iota * (_F * _M)
        iota_m = iota * _M
        iota_n = iota * _NMOVE
        neg1 = jnp.full((_L,), -1.0, jnp.float32)
        neginf = jnp.full((_L,), -jnp.inf, jnp.float32)

        def group(g, carry):
            pr_base = g * (_L * _F * _M)
            mw_base = g * (_L * _M)
            mk_base = g * (_L * _NMOVE)
            pr_idx = iota_f + pr_base
            runmax = neginf
            runidx = jnp.zeros((_L,), jnp.int32)
            for j in range(_NMOVE):
                m, k = divmod(j, 4)
                if k == 0:
                    act = plsc.load_gather(pr_v, [pr_idx + (m * _F + 1)]) == 1
                    mword = plsc.load_gather(mask_v,
                                             [iota_m + (mw_base + m)])
                off = m * _F + (_F - 8) + 2 * k
                tok = plsc.load_gather(pr_v, [pr_idx + off]) + 1
                tok = jnp.minimum(jnp.maximum(tok, 0), _V - 1)
                bp = plsc.load_gather(t3, [tok])
                mv = (mword & (1 << (8 * k))) != 0
                bpm = jnp.where(mv, bp, neg1)
                sc = jnp.where(act, bpm, neginf)
                upd = sc > runmax
                runmax = jnp.where(upd, sc, runmax)
                runidx = jnp.where(upd, jnp.full((_L,), j, jnp.int32), runidx)
                plsc.store_scatter(out_v, [iota_n + (mk_base + j)], bpm)
            idx_v[pl.ds(g * _L, _L)] = runidx
            return carry

        lax.fori_loop(0, ngroups, group, 0)

        pltpu.sync_copy(out_v, bp_hbm.at[pl.ds(wid * mkw, mkw)])
        pltpu.sync_copy(idx_v, idx_hbm.at[pl.ds(wid * bpw, bpw)])

    return sc_call


def kernel(private_reserve, move_mask, emb_table):
    sc_call = _build_sc_call()
    pr_flat = private_reserve.reshape(-1)
    mask_words = lax.bitcast_convert_type(
        move_mask.astype(jnp.uint8).reshape(_B * _M, 4), jnp.int32)
    emb_flat = emb_table.reshape(-1)
    bp_flat, index = sc_call(pr_flat, mask_words, emb_flat)
    return bp_flat.reshape(_B, _M, 4), index
